# baseline (speedup 1.0000x reference)
import jax
import jax.numpy as jnp
from jax.experimental import pallas as pl
from jax.experimental.pallas import tpu as pltpu

_B, _S, _J, _C, _TL = 16, 2048, 512, 10, 64
_TS = 1024
_NT_IN = _S // _TS
_NT = (_S + 1 + _TS - 1) // _TS
_W = 88


def _merge_window(base, rel0, sos_row, lab, nrows):
    if nrows == 65:
        blk = jnp.concatenate([sos_row, lab], axis=0)
    else:
        blk = jnp.concatenate([sos_row, lab, sos_row], axis=0)
    rows = jax.lax.broadcasted_iota(jnp.int32, (_W, nrows), 0) + rel0
    cols = jax.lax.broadcasted_iota(jnp.int32, (_W, nrows), 1)
    oh = (rows == cols).astype(jnp.float32)
    repl = jax.lax.dot_general(
        oh, blk, (((1,), (0,)), ((), ())),
        precision=jax.lax.Precision.HIGHEST,
        preferred_element_type=jnp.float32)
    rel = rel0 + jax.lax.broadcasted_iota(jnp.int32, (_W, 1), 0)
    mask = (rel >= 0) & (rel < nrows)
    return jnp.where(mask, repl, base)


def _body(lens_ref, c_ref, x_ref, sos_ref, lab_ref, tgt_hbm,
          ox_ref, ot_hbm, win_ref, carry_ref, sem1, sem2):
    b = pl.program_id(0)
    t = pl.program_id(1)
    lb = lens_ref[b]

    @pl.when((b == 0) & (t == 0))
    def _():
        for i in range(_B):
            pltpu.make_async_copy(tgt_hbm.at[i], ot_hbm.at[i],
                                  sem1.at[i]).start()

    xb = x_ref[0]

    @pl.when(t == 0)
    def _():
        carry_ref[...] = sos_ref[0]

    ox_ref[0, 0:1, :] = carry_ref[...]
    ox_ref[0, 1:_TS, :] = xb[:_TS - 1, :]
    carry_ref[...] = xb[_TS - 1:_TS, :]

    a = lb + 1 - t * _TS
    overlap = (lb + 66 > t * _TS) & (lb + 1 < t * _TS + _TS)

    @pl.when(overlap)
    def _():
        w = pl.multiple_of(jnp.clip((a // 8) * 8, 0, _TS - _W), 8)
        bw = ox_ref[0, pl.ds(w, _W), :]
        ox_ref[0, pl.ds(w, _W), :] = _merge_window(
            bw, w - a, sos_ref[0], lab_ref[0], 65)

    @pl.when(t == _NT - 1)
    def _():
        w8 = pl.multiple_of(jnp.clip((lb // 8) * 8, 0, _S - _W), 8)
        rcp = pltpu.make_async_copy(
            tgt_hbm.at[pl.ds(b, 1), pl.ds(w8, _W), :], win_ref, sem2)
        rcp.start()
        rcp.wait()
        win_ref[0] = _merge_window(
            win_ref[0], w8 - lb, sos_ref[0], lab_ref[0], 66)
        pltpu.make_async_copy(tgt_hbm.at[b], ot_hbm.at[b], sem1.at[b]).wait()
        wcp = pltpu.make_async_copy(
            win_ref, ot_hbm.at[pl.ds(b, 1), pl.ds(w8, _W), :], sem2)
        wcp.start()
        wcp.wait()


def kernel(x, tgt, lens, c, sos, labels):
    sos3 = sos[:, None, :]
    grid_spec = pltpu.PrefetchScalarGridSpec(
        num_scalar_prefetch=2,
        grid=(_B, _NT),
        in_specs=[
            pl.BlockSpec((1, _TS, _J),
                         lambda b, t, lens_ref, c_ref:
                         (b, jnp.minimum(t, _NT_IN - 1), 0)),
            pl.BlockSpec((1, 1, _J), lambda b, t, lens_ref, c_ref: (b, 0, 0)),
            pl.BlockSpec((1, _TL, _J),
                         lambda b, t, lens_ref, c_ref: (c_ref[b], 0, 0)),
            pl.BlockSpec(memory_space=pltpu.MemorySpace.HBM),
        ],
        out_specs=[
            pl.BlockSpec((1, _TS, _J),
                         lambda b, t, lens_ref, c_ref: (b, t, 0)),
            pl.BlockSpec(memory_space=pltpu.MemorySpace.HBM),
        ],
        scratch_shapes=[
            pltpu.VMEM((1, _W, _J), jnp.float32),
            pltpu.VMEM((1, _J), jnp.float32),
            pltpu.SemaphoreType.DMA((_B,)),
            pltpu.SemaphoreType.DMA,
        ],
    )
    out_x, out_tgt = pl.pallas_call(
        _body,
        grid_spec=grid_spec,
        out_shape=[
            jax.ShapeDtypeStruct((_B, _S + 1, _J), jnp.float32),
            jax.ShapeDtypeStruct((_B, _S, _J), jnp.float32),
        ],
    )(lens, c, x, sos3, labels, tgt)
    return (out_x, out_tgt, labels)


# speedup vs baseline: 12.0490x; 12.0490x over previous
import jax
import jax.numpy as jnp
from jax.experimental import pallas as pl
from jax.experimental.pallas import tpu as pltpu

_B, _S, _J, _C, _TL = 16, 2048, 512, 10, 64
_TS = 1024
_NT_IN = _S // _TS
_NT = (_S + 1 + _TS - 1) // _TS
_W = 88


def _merge_window(base, rel0, sos_row, lab, nrows):
    if nrows == 65:
        blk = jnp.concatenate([sos_row, lab], axis=0)
    else:
        blk = jnp.concatenate([sos_row, lab, sos_row], axis=0)
    rows = jax.lax.broadcasted_iota(jnp.int32, (_W, nrows), 0) + rel0
    cols = jax.lax.broadcasted_iota(jnp.int32, (_W, nrows), 1)
    oh = (rows == cols).astype(jnp.float32)
    repl = jax.lax.dot_general(
        oh, blk, (((1,), (0,)), ((), ())),
        precision=jax.lax.Precision.HIGHEST,
        preferred_element_type=jnp.float32)
    rel = rel0 + jax.lax.broadcasted_iota(jnp.int32, (_W, 1), 0)
    mask = (rel >= 0) & (rel < nrows)
    return jnp.where(mask, repl, base)


def _x_body(lens_ref, c_ref, x_ref, sos_ref, lab_ref, ox_ref, carry_ref):
    b = pl.program_id(0)
    t = pl.program_id(1)
    lb = lens_ref[b]
    xb = x_ref[0]

    @pl.when(t == 0)
    def _():
        carry_ref[...] = sos_ref[0]

    ox_ref[0, 0:1, :] = carry_ref[...]
    ox_ref[0, 1:_TS, :] = xb[:_TS - 1, :]
    carry_ref[...] = xb[_TS - 1:_TS, :]

    a = lb + 1 - t * _TS
    overlap = (lb + 66 > t * _TS) & (lb + 1 < t * _TS + _TS)

    @pl.when(overlap)
    def _():
        w = pl.multiple_of(jnp.clip((a // 8) * 8, 0, _TS - _W), 8)
        bw = ox_ref[0, pl.ds(w, _W), :]
        ox_ref[0, pl.ds(w, _W), :] = _merge_window(
            bw, w - a, sos_ref[0], lab_ref[0], 65)


def _tgt_body(lens_ref, c_ref, tgt_ref, sos_ref, lab_ref, ot_ref):
    b = pl.program_id(0)
    t = pl.program_id(1)
    lb = lens_ref[b]
    ot_ref[0] = tgt_ref[0]

    a = lb - t * _TS
    overlap = (lb + 66 > t * _TS) & (lb < t * _TS + _TS)

    @pl.when(overlap)
    def _():
        w = pl.multiple_of(jnp.clip((a // 8) * 8, 0, _TS - _W), 8)
        bw = tgt_ref[0, pl.ds(w, _W), :]
        ot_ref[0, pl.ds(w, _W), :] = _merge_window(
            bw, w - a, sos_ref[0], lab_ref[0], 66)


def kernel(x, tgt, lens, c, sos, labels):
    sos3 = sos[:, None, :]
    x_grid = pltpu.PrefetchScalarGridSpec(
        num_scalar_prefetch=2,
        grid=(_B, _NT),
        in_specs=[
            pl.BlockSpec((1, _TS, _J),
                         lambda b, t, lens_ref, c_ref:
                         (b, jnp.minimum(t, _NT_IN - 1), 0)),
            pl.BlockSpec((1, 1, _J), lambda b, t, lens_ref, c_ref: (b, 0, 0)),
            pl.BlockSpec((1, _TL, _J),
                         lambda b, t, lens_ref, c_ref: (c_ref[b], 0, 0)),
        ],
        out_specs=pl.BlockSpec((1, _TS, _J),
                               lambda b, t, lens_ref, c_ref: (b, t, 0)),
        scratch_shapes=[pltpu.VMEM((1, _J), jnp.float32)],
    )
    out_x = pl.pallas_call(
        _x_body,
        grid_spec=x_grid,
        out_shape=jax.ShapeDtypeStruct((_B, _S + 1, _J), jnp.float32),
    )(lens, c, x, sos3, labels)

    t_grid = pltpu.PrefetchScalarGridSpec(
        num_scalar_prefetch=2,
        grid=(_B, _NT_IN),
        in_specs=[
            pl.BlockSpec((1, _TS, _J),
                         lambda b, t, lens_ref, c_ref: (b, t, 0)),
            pl.BlockSpec((1, 1, _J), lambda b, t, lens_ref, c_ref: (b, 0, 0)),
            pl.BlockSpec((1, _TL, _J),
                         lambda b, t, lens_ref, c_ref: (c_ref[b], 0, 0)),
        ],
        out_specs=pl.BlockSpec((1, _TS, _J),
                               lambda b, t, lens_ref, c_ref: (b, t, 0)),
    )
    out_tgt = pl.pallas_call(
        _tgt_body,
        grid_spec=t_grid,
        out_shape=jax.ShapeDtypeStruct((_B, _S, _J), jnp.float32),
    )(lens, c, tgt, sos3, labels)

    return (out_x, out_tgt, labels)
